# baseline (speedup 1.0000x reference)
import functools

import jax
import jax.numpy as jnp
from jax import lax
from jax.experimental import pallas as pl
from jax.experimental.pallas import tpu as pltpu
from jax.experimental.pallas import tpu_sc as plsc

_NC = 2
_NS = 16
_NW = _NC * _NS
_L = 16
_B = 16384
_D = 64
_BPW = _B // _NW
_CW = 128
_NCH = _BPW // _CW


def _sc_body(pidx_hbm, widx_hbm, ptab_hbm, wtab_hbm, out_hbm,
             pidx_v, widx_v, prow_v, wrow_v, out_v, sem_p, sem_w):
    wid = lax.axis_index("s") * _NC + lax.axis_index("c")
    base = wid * _BPW

    for k in range(_NCH):
        pltpu.sync_copy(pidx_hbm.at[pl.ds(base + k * _CW, _CW)], pidx_v.at[k])
        pltpu.sync_copy(widx_hbm.at[pl.ds(base + k * _CW, _CW)], widx_v.at[k])

    copies = []
    for k in range(_NCH):
        copies.append(pltpu.async_copy(
            ptab_hbm.at[pidx_v.at[k]], prow_v.at[pl.ds(k * _CW, _CW)], sem_p))
        copies.append(pltpu.async_copy(
            wtab_hbm.at[widx_v.at[k]], wrow_v.at[pl.ds(k * _CW, _CW)], sem_w))
    for c in copies:
        c.wait()

    lane = lax.iota(jnp.int32, _L)

    def group(g, carry):
        res = jnp.zeros((_L,), jnp.float32)
        for rl in range(_L):
            r = g * _L + rl
            acc = jnp.zeros((_L,), jnp.float32)
            for j in range(_D // _L):
                a = prow_v[r, pl.ds(j * _L, _L)]
                b = wrow_v[r, pl.ds(j * _L, _L)]
                acc = acc + a * b
            s = lax.reduce_sum_p.bind(acc, axes=(0,))
            res = jnp.where(lane == rl, s, res)
        out_v[pl.ds(g * _L, _L)] = 1.0 / (1.0 + jnp.exp(-res))
        return carry

    lax.fori_loop(0, _BPW // _L, group, 0)

    pltpu.sync_copy(out_v, out_hbm.at[pl.ds(base, _BPW)])


@jax.jit
def kernel(place_indices, word_indices, place_table, word_table):
    mesh = plsc.VectorSubcoreMesh(core_axis_name="c", subcore_axis_name="s",
                                  num_cores=_NC, num_subcores=_NS)
    f = pl.kernel(
        _sc_body,
        out_type=jax.ShapeDtypeStruct((_B,), jnp.float32),
        mesh=mesh,
        scratch_types=[
            pltpu.VMEM((_NCH, _CW), jnp.int32),
            pltpu.VMEM((_NCH, _CW), jnp.int32),
            pltpu.VMEM((_BPW, _D), jnp.float32),
            pltpu.VMEM((_BPW, _D), jnp.float32),
            pltpu.VMEM((_BPW,), jnp.float32),
            pltpu.SemaphoreType.DMA,
            pltpu.SemaphoreType.DMA,
        ],
        compiler_params=pltpu.CompilerParams(
            needs_layout_passes=False, use_tc_tiling_on_sc=False),
    )
    return f(place_indices, word_indices, place_table, word_table)
